# speedup vs baseline: 3.7142x; 3.7142x over previous
import functools

import jax
import jax.numpy as jnp
from jax import lax
from jax.experimental import pallas as pl
from jax.experimental.pallas import tpu as pltpu
from jax.experimental.pallas import tpu_sc as plsc

N = 10000
D = 128
E = 320000
NC = 2
NS = 16
L = 16

B = 128
NBLK = E // B
BLK_ITERS = (NBLK + NS - 1) // NS

HALF = N // NC
ACC_ROWS = 5120
ZROWS_PER_TILE = ACC_ROWS // NS
OUT_CHUNK = 8
NOCHUNK = HALF // OUT_CHUNK
OUT_ITERS = (NOCHUNK + NS - 1) // NS


def _body(x_hbm, row_hbm, col_hbm, out_hbm,
          acc, zbuf, row_blk, col_blk, ccol_blk, rows_v, sem):
    c = lax.axis_index("c")
    s = lax.axis_index("s")
    lo = c * HALF

    def zero_zbuf(k, _):
        r = k // (D // L)
        col = (k % (D // L)) * L
        zbuf[r, pl.ds(col, L)] = jnp.zeros((L,), jnp.float32)
        return 0
    lax.fori_loop(0, 16 * (D // L), zero_zbuf, 0)

    def zero_acc(k, _):
        pltpu.sync_copy(zbuf, acc.at[pl.ds(s * ZROWS_PER_TILE + k * 16, 16)])
        return 0
    lax.fori_loop(0, ZROWS_PER_TILE // 16, zero_acc, 0)

    plsc.subcore_barrier()

    def edge_block(i, _):
        b = s + i * NS

        @pl.when(b < NBLK)
        def _():
            base = b * B
            pltpu.sync_copy(row_hbm.at[pl.ds(base, B)], row_blk)
            pltpu.sync_copy(col_hbm.at[pl.ds(base, B)], col_blk)

            def remap(j, _):
                colv = col_blk[pl.ds(j * L, L)]
                rel = colv - lo
                m = (rel >= 0) & (rel < HALF)
                ccol_blk[pl.ds(j * L, L)] = jnp.where(m, rel, HALF)
                return 0
            lax.fori_loop(0, B // L, remap, 0)

            pltpu.async_copy(x_hbm.at[row_blk], rows_v, sem).wait()
            pltpu.sync_copy(rows_v, acc.at[ccol_blk], add=True)
        return 0
    lax.fori_loop(0, BLK_ITERS, edge_block, 0)

    plsc.subcore_barrier()

    def out_chunk(i, _):
        ch = s + i * NS

        @pl.when(ch < NOCHUNK)
        def _():
            r = ch * OUT_CHUNK
            pltpu.sync_copy(acc.at[pl.ds(r, OUT_CHUNK)],
                            out_hbm.at[pl.ds(lo + r, OUT_CHUNK)])
        return 0
    lax.fori_loop(0, OUT_ITERS, out_chunk, 0)


@jax.jit
def kernel(x, edge_index):
    row = edge_index[0]
    col = edge_index[1]
    mesh = plsc.VectorSubcoreMesh(core_axis_name="c", subcore_axis_name="s",
                                  num_cores=NC, num_subcores=NS)
    f = pl.kernel(
        _body,
        out_type=jax.ShapeDtypeStruct((N, D), jnp.float32),
        mesh=mesh,
        scratch_types=[
            pltpu.VMEM_SHARED((ACC_ROWS, D), jnp.float32),
            pltpu.VMEM((16, D), jnp.float32),
            pltpu.VMEM((B,), jnp.int32),
            pltpu.VMEM((B,), jnp.int32),
            pltpu.VMEM((B,), jnp.int32),
            pltpu.VMEM((B, D), jnp.float32),
            pltpu.SemaphoreType.DMA,
        ],
    )
    return f(x, row, col)


# speedup vs baseline: 11.2518x; 3.0294x over previous
import jax
import jax.numpy as jnp
from jax import lax
from jax.experimental import pallas as pl
from jax.experimental.pallas import tpu as pltpu
from jax.experimental.pallas import tpu_sc as plsc

N = 10000
D = 128
E = 320000
NC = 2
NS = 16
L = 16

B = 128
EPT = E // (NC * NS)
NBLK = EPT // B
TAIL = EPT - NBLK * B
NPAIR = NBLK // 2 - 1

ACC_ROWS = 10240
ZCH = 16
ZITER = ACC_ROWS // NS // ZCH
NOCHUNK = N // ZCH
OUT_ITERS = (NOCHUNK + NS - 1) // NS


def _sc_body(x_hbm, row_hbm, col_hbm, part_hbm,
             acc, zbuf, row_all, col_b0, col_b1, col_tail,
             rows_v0, rows_v1, rows_tail,
             sem_c0, sem_c1, sem_g0, sem_g1, sem_t):
    c = lax.axis_index("c")
    s = lax.axis_index("s")
    wbase = (c * NS + s) * EPT

    def zero_zbuf(k, _):
        r = k // (D // L)
        cc = (k % (D // L)) * L
        zbuf[r, pl.ds(cc, L)] = jnp.zeros((L,), jnp.float32)
        return 0
    lax.fori_loop(0, ZCH * (D // L), zero_zbuf, 0)

    def zero_acc(k, _):
        pltpu.sync_copy(zbuf, acc.at[pl.ds((s * ZITER + k) * ZCH, ZCH)])
        return 0
    lax.fori_loop(0, ZITER, zero_acc, 0)

    pltpu.sync_copy(row_hbm.at[pl.ds(wbase, EPT)], row_all)

    plsc.subcore_barrier()

    def start_col(j, col_b, sem):
        pltpu.async_copy(col_hbm.at[pl.ds(wbase + j * B, B)], col_b, sem)

    def wait_col(j, col_b, sem):
        pltpu.make_async_copy(col_hbm.at[pl.ds(wbase + j * B, B)], col_b,
                              sem).wait()

    def start_gather(j, rows_v, sem):
        pltpu.async_copy(x_hbm.at[row_all.at[pl.ds(j * B, B)]], rows_v, sem)

    def wait_gather(j, rows_v, sem):
        pltpu.make_async_copy(x_hbm.at[row_all.at[pl.ds(j * B, B)]], rows_v,
                              sem).wait()

    start_col(0, col_b0, sem_c0)
    start_gather(0, rows_v0, sem_g0)
    start_col(1, col_b1, sem_c1)
    start_gather(1, rows_v1, sem_g1)

    def pair(i, _):
        j0 = 2 * i
        wait_col(j0, col_b0, sem_c0)
        wait_gather(j0, rows_v0, sem_g0)
        pltpu.sync_copy(rows_v0, acc.at[col_b0], add=True)
        start_col(j0 + 2, col_b0, sem_c0)
        start_gather(j0 + 2, rows_v0, sem_g0)

        wait_col(j0 + 1, col_b1, sem_c1)
        wait_gather(j0 + 1, rows_v1, sem_g1)
        pltpu.sync_copy(rows_v1, acc.at[col_b1], add=True)
        start_col(j0 + 3, col_b1, sem_c1)
        start_gather(j0 + 3, rows_v1, sem_g1)
        return 0
    lax.fori_loop(0, NPAIR, pair, 0)

    wait_col(NBLK - 2, col_b0, sem_c0)
    wait_gather(NBLK - 2, rows_v0, sem_g0)
    pltpu.sync_copy(rows_v0, acc.at[col_b0], add=True)
    wait_col(NBLK - 1, col_b1, sem_c1)
    wait_gather(NBLK - 1, rows_v1, sem_g1)
    pltpu.sync_copy(rows_v1, acc.at[col_b1], add=True)

    tbase = wbase + NBLK * B
    pltpu.sync_copy(col_hbm.at[pl.ds(tbase, TAIL)], col_tail)
    pltpu.async_copy(x_hbm.at[row_all.at[pl.ds(NBLK * B, TAIL)]],
                     rows_tail, sem_t).wait()
    pltpu.sync_copy(rows_tail, acc.at[col_tail], add=True)

    plsc.subcore_barrier()

    def out_chunk(i, _):
        ch = s + i * NS

        @pl.when(ch < NOCHUNK)
        def _():
            r = ch * ZCH
            pltpu.sync_copy(acc.at[pl.ds(r, ZCH)],
                            part_hbm.at[c, pl.ds(r, ZCH)])
        return 0
    lax.fori_loop(0, OUT_ITERS, out_chunk, 0)


def _add_body(a_ref, b_ref, o_ref):
    o_ref[...] = a_ref[...] + b_ref[...]


@jax.jit
def kernel(x, edge_index):
    row = edge_index[0]
    col = edge_index[1]
    mesh = plsc.VectorSubcoreMesh(core_axis_name="c", subcore_axis_name="s",
                                  num_cores=NC, num_subcores=NS)
    sc = pl.kernel(
        _sc_body,
        out_type=jax.ShapeDtypeStruct((NC, N, D), jnp.float32),
        mesh=mesh,
        scratch_types=[
            pltpu.VMEM_SHARED((ACC_ROWS, D), jnp.float32),
            pltpu.VMEM((ZCH, D), jnp.float32),
            pltpu.VMEM((EPT,), jnp.int32),
            pltpu.VMEM((B,), jnp.int32),
            pltpu.VMEM((B,), jnp.int32),
            pltpu.VMEM((TAIL,), jnp.int32),
            pltpu.VMEM((B, D), jnp.float32),
            pltpu.VMEM((B, D), jnp.float32),
            pltpu.VMEM((TAIL, D), jnp.float32),
            pltpu.SemaphoreType.DMA,
            pltpu.SemaphoreType.DMA,
            pltpu.SemaphoreType.DMA,
            pltpu.SemaphoreType.DMA,
            pltpu.SemaphoreType.DMA,
        ],
    )
    part = sc(x, row, col)

    grid = 10
    out = pl.pallas_call(
        _add_body,
        out_shape=jax.ShapeDtypeStruct((N, D), jnp.float32),
        grid=(grid,),
        in_specs=[
            pl.BlockSpec((N // grid, D), lambda i: (i, 0)),
            pl.BlockSpec((N // grid, D), lambda i: (i, 0)),
        ],
        out_specs=pl.BlockSpec((N // grid, D), lambda i: (i, 0)),
    )(part[0], part[1])
    return out
